# Initial kernel scaffold; baseline (speedup 1.0000x reference)
import jax
import jax.numpy as jnp
from jax.experimental import pallas as pl


def kernel(x, emb_0, emb_1, emb_2, emb_3, emb_4, emb_5, emb_6, emb_7, emb_8, emb_9, emb_10, emb_11, W_lin, b_lin, W_fin, b_fin):
    raise NotImplementedError("write your pallas kernel here")


# speedup vs baseline: 23.0564x; 23.0564x over previous
import functools

import numpy as np
import jax
import jax.numpy as jnp
from jax.experimental import pallas as pl

FDIMS = [20, 38, 119, 4, 12, 12, 10, 6, 6, 2, 2, 4]
NC = len(FDIMS)
OFFS = np.concatenate([[0], np.cumsum(FDIMS)]).astype(np.int32)
VOCAB = int(OFFS[-1])
VPAD = 256
EMB = 128
SIGMA = 32
PEP = 16
BLK = 2048


def _prep_body(t_ref, wlin_ref, wfin_ref, blin_ref, bfin_ref,
               tp_ref, w48_ref, bp_ref):
    wf1 = wfin_ref[0:EMB, :]
    tp_ref[...] = jnp.dot(t_ref[...], wf1, preferred_element_type=jnp.float32)
    w48_ref[0:SIGMA, :] = jnp.dot(wlin_ref[...], wf1,
                                  preferred_element_type=jnp.float32)
    w48_ref[SIGMA:SIGMA + PEP, :] = wfin_ref[EMB:EMB + PEP, :]
    bp_ref[...] = (jnp.dot(blin_ref[...], wf1,
                           preferred_element_type=jnp.float32)
                   + bfin_ref[...])


def _main_body(x_ref, e_ref, lt_ref, tp_ref, w48_ref, bp_ref, out_ref):
    xb = x_ref[...]
    g = jnp.dot(xb[:, 0:16], e_ref[...], preferred_element_type=jnp.float32)
    s = (g == lt_ref[0:1, :]).astype(jnp.float32)
    acc = jnp.dot(s, tp_ref[...], preferred_element_type=jnp.float32)
    acc = acc + jnp.dot(xb[:, NC:NC + SIGMA + PEP], w48_ref[...],
                        preferred_element_type=jnp.float32)
    out_ref[...] = acc + bp_ref[0:1, :]


def _static_consts():
    e = np.zeros((16, VPAD), np.float32)
    lt = np.full((VPAD,), -1.0, np.float32)
    for i in range(NC):
        lo, hi = int(OFFS[i]), int(OFFS[i + 1])
        e[i, lo:hi] = 1.0
        lt[lo:hi] = np.arange(hi - lo, dtype=np.float32)
    return jnp.asarray(e), jnp.asarray(np.broadcast_to(lt, (8, VPAD)).copy())


@functools.partial(jax.jit, static_argnums=())
def kernel(x, emb_0, emb_1, emb_2, emb_3, emb_4, emb_5, emb_6, emb_7, emb_8,
           emb_9, emb_10, emb_11, W_lin, b_lin, W_fin, b_fin):
    n = x.shape[0]
    tables = [emb_0, emb_1, emb_2, emb_3, emb_4, emb_5, emb_6, emb_7, emb_8,
              emb_9, emb_10, emb_11]
    t = jnp.concatenate(tables, axis=0)
    t = jnp.pad(t, ((0, VPAD - VOCAB), (0, 0)))
    blin8 = jnp.broadcast_to(b_lin[None, :], (8, EMB))
    bfin8 = jnp.broadcast_to(b_fin[None, :], (8, EMB))

    tp, w48, bp = pl.pallas_call(
        _prep_body,
        out_shape=(
            jax.ShapeDtypeStruct((VPAD, EMB), jnp.float32),
            jax.ShapeDtypeStruct((SIGMA + PEP, EMB), jnp.float32),
            jax.ShapeDtypeStruct((8, EMB), jnp.float32),
        ),
    )(t, W_lin, W_fin, blin8, bfin8)

    e16, lt = _static_consts()
    grid = (pl.cdiv(n, BLK),)
    out = pl.pallas_call(
        _main_body,
        grid=grid,
        in_specs=[
            pl.BlockSpec((BLK, x.shape[1]), lambda i: (i, 0)),
            pl.BlockSpec((16, VPAD), lambda i: (0, 0)),
            pl.BlockSpec((8, VPAD), lambda i: (0, 0)),
            pl.BlockSpec((VPAD, EMB), lambda i: (0, 0)),
            pl.BlockSpec((SIGMA + PEP, EMB), lambda i: (0, 0)),
            pl.BlockSpec((8, EMB), lambda i: (0, 0)),
        ],
        out_specs=pl.BlockSpec((BLK, EMB), lambda i: (i, 0)),
        out_shape=jax.ShapeDtypeStruct((n, EMB), jnp.float32),
    )(x, e16, lt, tp, w48, bp)
    return out


# speedup vs baseline: 23.1276x; 1.0031x over previous
import functools

import numpy as np
import jax
import jax.numpy as jnp
from jax.experimental import pallas as pl

FDIMS = [20, 38, 119, 4, 12, 12, 10, 6, 6, 2, 2, 4]
NC = len(FDIMS)
OFFS = np.concatenate([[0], np.cumsum(FDIMS)]).astype(np.int32)
VOCAB = int(OFFS[-1])
VPAD = 256
EMB = 128
SIGMA = 32
PEP = 16
BLK = 2048


def _prep_body(t_ref, wlin_ref, wfin_ref, blin_ref, bfin_ref,
               tp_ref, w48_ref, bp_ref):
    wf1 = wfin_ref[0:EMB, :]
    tp_ref[...] = jnp.dot(t_ref[...], wf1,
                          preferred_element_type=jnp.float32
                          ).astype(jnp.bfloat16)
    w48_ref[0:SIGMA, :] = jnp.dot(wlin_ref[...], wf1,
                                  preferred_element_type=jnp.float32
                                  ).astype(jnp.bfloat16)
    w48_ref[SIGMA:SIGMA + PEP, :] = wfin_ref[EMB:EMB + PEP, :].astype(
        jnp.bfloat16)
    bp_ref[...] = (jnp.dot(blin_ref[...], wf1,
                           preferred_element_type=jnp.float32)
                   + bfin_ref[...])


def _main_body(x_ref, e_ref, lt_ref, tp_ref, w48_ref, bp_ref, out_ref):
    xb = x_ref[...].astype(jnp.bfloat16)
    g = jnp.dot(xb[:, 0:16], e_ref[...], preferred_element_type=jnp.float32)
    s = (g == lt_ref[0:1, :]).astype(jnp.bfloat16)
    acc = jnp.dot(s, tp_ref[...], preferred_element_type=jnp.float32)
    acc = acc + jnp.dot(xb[:, NC:NC + SIGMA + PEP], w48_ref[...],
                        preferred_element_type=jnp.float32)
    out_ref[...] = acc + bp_ref[0:1, :]


def _static_consts():
    e = np.zeros((16, VPAD), np.float32)
    lt = np.full((VPAD,), -1.0, np.float32)
    for i in range(NC):
        lo, hi = int(OFFS[i]), int(OFFS[i + 1])
        e[i, lo:hi] = 1.0
        lt[lo:hi] = np.arange(hi - lo, dtype=np.float32)
    return (jnp.asarray(e, dtype=jnp.bfloat16),
            jnp.asarray(np.broadcast_to(lt, (8, VPAD)).copy()))


@functools.partial(jax.jit, static_argnums=())
def kernel(x, emb_0, emb_1, emb_2, emb_3, emb_4, emb_5, emb_6, emb_7, emb_8,
           emb_9, emb_10, emb_11, W_lin, b_lin, W_fin, b_fin):
    n = x.shape[0]
    tables = [emb_0, emb_1, emb_2, emb_3, emb_4, emb_5, emb_6, emb_7, emb_8,
              emb_9, emb_10, emb_11]
    t = jnp.concatenate(tables, axis=0)
    t = jnp.pad(t, ((0, VPAD - VOCAB), (0, 0)))
    blin8 = jnp.broadcast_to(b_lin[None, :], (8, EMB))
    bfin8 = jnp.broadcast_to(b_fin[None, :], (8, EMB))

    tp, w48, bp = pl.pallas_call(
        _prep_body,
        out_shape=(
            jax.ShapeDtypeStruct((VPAD, EMB), jnp.bfloat16),
            jax.ShapeDtypeStruct((SIGMA + PEP, EMB), jnp.bfloat16),
            jax.ShapeDtypeStruct((8, EMB), jnp.float32),
        ),
    )(t, W_lin, W_fin, blin8, bfin8)

    e16, lt = _static_consts()
    grid = (pl.cdiv(n, BLK),)
    out = pl.pallas_call(
        _main_body,
        grid=grid,
        in_specs=[
            pl.BlockSpec((BLK, x.shape[1]), lambda i: (i, 0)),
            pl.BlockSpec((16, VPAD), lambda i: (0, 0)),
            pl.BlockSpec((8, VPAD), lambda i: (0, 0)),
            pl.BlockSpec((VPAD, EMB), lambda i: (0, 0)),
            pl.BlockSpec((SIGMA + PEP, EMB), lambda i: (0, 0)),
            pl.BlockSpec((8, EMB), lambda i: (0, 0)),
        ],
        out_specs=pl.BlockSpec((BLK, EMB), lambda i: (i, 0)),
        out_shape=jax.ShapeDtypeStruct((n, EMB), jnp.float32),
    )(x, e16, lt, tp, w48, bp)
    return out
